# baseline (speedup 1.0000x reference)
import jax
import jax.numpy as jnp
from jax import lax
from jax.experimental import pallas as pl
from jax.experimental.pallas import tpu as pltpu, tpu_sc as plsc

B, S, W = 4, 2048, 1024
TOKEN_TYPES = 16
NUM_WORKERS = 32
S_PER_W = S // NUM_WORKERS
CHUNK = 16
QUADS = S_PER_W // CHUNK
STEPS = QUADS * B
LANES = 16
W_CHUNKS = W // LANES


def _body(in_hbm, ids_hbm, table_hbm, pos_hbm, out_hbm,
          acc0, acc1, acc2, acc3, pos0, pos1, tblb, ids_v,
          sem_in, sem_pos, sem_out):
    accs = (acc0, acc1, acc2, acc3)
    poss = (pos0, pos1)
    nc = plsc.get_sparse_core_info().num_cores
    wid = lax.axis_index("s") * nc + lax.axis_index("c")
    s0 = wid * S_PER_W

    pltpu.sync_copy(table_hbm, tblb)
    for b in range(B):
        pltpu.sync_copy(ids_hbm.at[b, pl.ds(s0, S_PER_W)], ids_v.at[b])

    in_d = [None] * STEPS
    pos_d = [None] * QUADS
    out_d = [None] * STEPS

    def issue_in(k):
        q, b = divmod(k, B)
        in_d[k] = pltpu.async_copy(
            in_hbm.at[b, pl.ds(s0 + q * CHUNK, CHUNK)], accs[k % 4], sem_in)

    def issue_pos(q):
        pos_d[q] = pltpu.async_copy(
            pos_hbm.at[pl.ds(s0 + q * CHUNK, CHUNK)], poss[q % 2], sem_pos)

    issue_pos(0)
    issue_pos(1)
    issue_in(0)
    issue_in(1)
    for k in range(STEPS):
        q, b = divmod(k, B)
        if k >= 2:
            out_d[k - 2].wait()
        if k + 2 < STEPS:
            issue_in(k + 2)
        in_d[k].wait()
        if b == 0:
            pos_d[q].wait()
        acc, pos = accs[k % 4], poss[q % 2]
        qoff = q * CHUNK

        idvec = ids_v[b, pl.ds(qoff, LANES)]
        trows = [idvec[r] for r in range(CHUNK)]

        def col_add(j, carry):
            sl = pl.ds(j * LANES, LANES)
            for r in range(CHUNK):
                plsc.addupdate(acc.at[r, sl], tblb[trows[r], sl] + pos[r, sl])
            return carry

        lax.fori_loop(0, W_CHUNKS, col_add, 0)
        out_d[k] = pltpu.async_copy(
            acc, out_hbm.at[b, pl.ds(s0 + qoff, CHUNK)], sem_out)
        if b == B - 1 and q + 2 < QUADS:
            issue_pos(q + 2)
    out_d[STEPS - 2].wait()
    out_d[STEPS - 1].wait()


@jax.jit
def kernel(inputs, token_type_ids, token_type_table, full_position_embeddings):
    ids = token_type_ids.astype(jnp.int32)
    run = pl.kernel(
        _body,
        out_type=jax.ShapeDtypeStruct((B, S, W), jnp.float32),
        mesh=plsc.VectorSubcoreMesh(core_axis_name="c", subcore_axis_name="s"),
        scratch_types=[
            pltpu.VMEM((CHUNK, W), jnp.float32),
            pltpu.VMEM((CHUNK, W), jnp.float32),
            pltpu.VMEM((CHUNK, W), jnp.float32),
            pltpu.VMEM((CHUNK, W), jnp.float32),
            pltpu.VMEM((CHUNK, W), jnp.float32),
            pltpu.VMEM((CHUNK, W), jnp.float32),
            pltpu.VMEM((TOKEN_TYPES, W), jnp.float32),
            pltpu.VMEM((B, S_PER_W), jnp.int32),
            pltpu.SemaphoreType.DMA,
            pltpu.SemaphoreType.DMA,
            pltpu.SemaphoreType.DMA,
        ],
    )
    return run(inputs, ids, token_type_table, full_position_embeddings)


# speedup vs baseline: 1.7107x; 1.7107x over previous
import jax
import jax.numpy as jnp
from jax import lax
from jax.experimental import pallas as pl
from jax.experimental.pallas import tpu as pltpu, tpu_sc as plsc

B, S, W = 4, 2048, 1024
TOKEN_TYPES = 16
NUM_WORKERS = 32
S_PER_W = S // NUM_WORKERS
CHUNK = 16
QUADS = S_PER_W // CHUNK
STEPS = QUADS * B
LANES = 16
W_CHUNKS = W // LANES


def _body(in_hbm, ids_hbm, table_hbm, pos_hbm, out_hbm,
          acc0, acc1, acc2, acc3, pos0, pos1, tblb, ids_v,
          sem_in, sem_pos, sem_out):
    accs = (acc0, acc1, acc2, acc3)
    poss = (pos0, pos1)
    nc = plsc.get_sparse_core_info().num_cores
    wid = lax.axis_index("s") * nc + lax.axis_index("c")
    s0 = wid * S_PER_W

    pltpu.sync_copy(table_hbm, tblb)
    for b in range(B):
        pltpu.sync_copy(ids_hbm.at[b, pl.ds(s0, S_PER_W)], ids_v.at[b])

    in_d = [None] * STEPS
    pos_d = [None] * QUADS
    out_d = [None] * STEPS

    def issue_in(k):
        q, b = divmod(k, B)
        in_d[k] = pltpu.async_copy(
            in_hbm.at[b, pl.ds(s0 + q * CHUNK, CHUNK)], accs[k % 4], sem_in)

    def issue_pos(q):
        pos_d[q] = pltpu.async_copy(
            pos_hbm.at[pl.ds(s0 + q * CHUNK, CHUNK)], poss[q % 2], sem_pos)

    issue_pos(0)
    issue_pos(1)
    issue_in(0)
    issue_in(1)
    for k in range(STEPS):
        q, b = divmod(k, B)
        if k >= 2:
            out_d[k - 2].wait()
        if k + 2 < STEPS:
            issue_in(k + 2)
        in_d[k].wait()
        if b == 0:
            pos_d[q].wait()
        acc, pos = accs[k % 4], poss[q % 2]
        qoff = q * CHUNK

        idvec = ids_v[b, pl.ds(qoff, LANES)]
        trows = [idvec[r] for r in range(CHUNK)]

        @plsc.parallel_loop(0, W_CHUNKS, unroll=2)
        def col_add(j):
            sl = pl.ds(j * LANES, LANES)
            for r in range(CHUNK):
                plsc.addupdate(acc.at[r, sl], tblb[trows[r], sl] + pos[r, sl])
        out_d[k] = pltpu.async_copy(
            acc, out_hbm.at[b, pl.ds(s0 + qoff, CHUNK)], sem_out)
        if b == B - 1 and q + 2 < QUADS:
            issue_pos(q + 2)
    out_d[STEPS - 2].wait()
    out_d[STEPS - 1].wait()


@jax.jit
def kernel(inputs, token_type_ids, token_type_table, full_position_embeddings):
    ids = token_type_ids.astype(jnp.int32)
    run = pl.kernel(
        _body,
        out_type=jax.ShapeDtypeStruct((B, S, W), jnp.float32),
        mesh=plsc.VectorSubcoreMesh(core_axis_name="c", subcore_axis_name="s"),
        scratch_types=[
            pltpu.VMEM((CHUNK, W), jnp.float32),
            pltpu.VMEM((CHUNK, W), jnp.float32),
            pltpu.VMEM((CHUNK, W), jnp.float32),
            pltpu.VMEM((CHUNK, W), jnp.float32),
            pltpu.VMEM((CHUNK, W), jnp.float32),
            pltpu.VMEM((CHUNK, W), jnp.float32),
            pltpu.VMEM((TOKEN_TYPES, W), jnp.float32),
            pltpu.VMEM((B, S_PER_W), jnp.int32),
            pltpu.SemaphoreType.DMA,
            pltpu.SemaphoreType.DMA,
            pltpu.SemaphoreType.DMA,
        ],
    )
    return run(inputs, ids, token_type_table, full_position_embeddings)


# speedup vs baseline: 1.7463x; 1.0208x over previous
import jax
import jax.numpy as jnp
from jax import lax
from jax.experimental import pallas as pl
from jax.experimental.pallas import tpu as pltpu, tpu_sc as plsc

B, S, W = 4, 2048, 1024
TOKEN_TYPES = 16
NUM_WORKERS = 32
S_PER_W = S // NUM_WORKERS
CHUNK = 16
QUADS = S_PER_W // CHUNK
STEPS = QUADS * B
LANES = 16
W_CHUNKS = W // LANES


def _body(in_hbm, ids_hbm, table_hbm, pos_hbm, out_hbm,
          acc0, acc1, acc2, acc3, pos0, pos1, tblb, ids_v,
          sem_in, sem_pos, sem_out, sem_misc):
    accs = (acc0, acc1, acc2, acc3)
    poss = (pos0, pos1)
    nc = plsc.get_sparse_core_info().num_cores
    wid = lax.axis_index("s") * nc + lax.axis_index("c")
    s0 = wid * S_PER_W

    tbl_cp = pltpu.async_copy(table_hbm, tblb, sem_misc)
    ids_cp = [
        pltpu.async_copy(ids_hbm.at[b, pl.ds(s0, S_PER_W)], ids_v.at[b],
                         sem_misc)
        for b in range(B)
    ]

    in_d = [None] * STEPS
    pos_d = [None] * QUADS
    out_d = [None] * STEPS

    def issue_in(k):
        q, b = divmod(k, B)
        in_d[k] = pltpu.async_copy(
            in_hbm.at[b, pl.ds(s0 + q * CHUNK, CHUNK)], accs[k % 4], sem_in)

    def issue_pos(q):
        pos_d[q] = pltpu.async_copy(
            pos_hbm.at[pl.ds(s0 + q * CHUNK, CHUNK)], poss[q % 2], sem_pos)

    issue_pos(0)
    issue_pos(1)
    issue_in(0)
    issue_in(1)
    tbl_cp.wait()
    for cp in ids_cp:
        cp.wait()
    for k in range(STEPS):
        q, b = divmod(k, B)
        if k >= 2:
            out_d[k - 2].wait()
        if k + 2 < STEPS:
            issue_in(k + 2)
        in_d[k].wait()
        if b == 0:
            pos_d[q].wait()
        acc, pos = accs[k % 4], poss[q % 2]
        qoff = q * CHUNK

        idvec = ids_v[b, pl.ds(qoff, LANES)]
        trows = [idvec[r] for r in range(CHUNK)]

        @plsc.parallel_loop(0, W_CHUNKS, unroll=2)
        def col_add(j):
            sl = pl.ds(j * LANES, LANES)
            for r in range(CHUNK):
                plsc.addupdate(acc.at[r, sl], tblb[trows[r], sl] + pos[r, sl])
        out_d[k] = pltpu.async_copy(
            acc, out_hbm.at[b, pl.ds(s0 + qoff, CHUNK)], sem_out)
        if b == B - 1 and q + 2 < QUADS:
            issue_pos(q + 2)
    out_d[STEPS - 2].wait()
    out_d[STEPS - 1].wait()


@jax.jit
def kernel(inputs, token_type_ids, token_type_table, full_position_embeddings):
    ids = token_type_ids.astype(jnp.int32)
    run = pl.kernel(
        _body,
        out_type=jax.ShapeDtypeStruct((B, S, W), jnp.float32),
        mesh=plsc.VectorSubcoreMesh(core_axis_name="c", subcore_axis_name="s"),
        scratch_types=[
            pltpu.VMEM((CHUNK, W), jnp.float32),
            pltpu.VMEM((CHUNK, W), jnp.float32),
            pltpu.VMEM((CHUNK, W), jnp.float32),
            pltpu.VMEM((CHUNK, W), jnp.float32),
            pltpu.VMEM((CHUNK, W), jnp.float32),
            pltpu.VMEM((CHUNK, W), jnp.float32),
            pltpu.VMEM((TOKEN_TYPES, W), jnp.float32),
            pltpu.VMEM((B, S_PER_W), jnp.int32),
            pltpu.SemaphoreType.DMA,
            pltpu.SemaphoreType.DMA,
            pltpu.SemaphoreType.DMA,
            pltpu.SemaphoreType.DMA,
        ],
    )
    return run(inputs, ids, token_type_table, full_position_embeddings)
